# Initial kernel scaffold; baseline (speedup 1.0000x reference)
import jax
import jax.numpy as jnp
from jax.experimental import pallas as pl


def kernel(x, conv_x_w, conv_x_b, conv_y_w, conv_y_b, W1_w, W1_b, W2_w, W2_b):
    raise NotImplementedError("write your pallas kernel here")


# speedup vs baseline: 26.8531x; 26.8531x over previous
import functools

import jax
import jax.numpy as jnp
from jax.experimental import pallas as pl
from jax.experimental.pallas import tpu as pltpu

K1 = 10
R = 256

def _batcher(n):
    pairs = []
    t = (n - 1).bit_length()
    p = 1 << (t - 1)
    while p > 0:
        q = 1 << (t - 1)
        r = 0
        d = p
        while d > 0:
            for i in range(n - d):
                if (i & p) == r:
                    pairs.append((i, i + d))
            d = q - p
            q >>= 1
            r = p
        p >>= 1
    return pairs


_SORT10 = _batcher(K1)


def _knn_body(xT_ref, xa_ref, m_ref, bias_ref, out_ref):
    n = xa_ref.shape[1]
    xa = xa_ref[0]
    xax = xa[:, 0:1]
    xay = xa[:, 1:2]
    xsT = xT_ref[0]
    sq_all = xax * xax + xay * xay
    sq_tile = xsT[0:1, :] ** 2 + xsT[1:2, :] ** 2

    prod = jax.lax.dot_general(xa, xsT, (((1,), (0,)), ((), ())),
                               preferred_element_type=jnp.float32)
    d2 = jnp.maximum(sq_all + sq_tile - 2.0 * prod, 0.0)

    iota = jax.lax.broadcasted_iota(jnp.int32, (n, R), 0)

    xs_rows = []
    ys_rows = []
    for _ in range(K1):
        m = jnp.min(d2, axis=0)
        eq = d2 == m[None, :]
        idx = jnp.max(jnp.where(eq, iota, -1), axis=0)
        onehot = eq & (iota == idx[None, :])
        d2 = jnp.where(onehot, float("inf"), d2)
        sel = onehot.astype(jnp.float32)
        xs_rows.append(jnp.sum(sel * xax, axis=0))
        ys_rows.append(jnp.sum(sel * xay, axis=0))

    def net(kx, ky):
        kx = list(kx)
        ky = list(ky)
        for i, j in _SORT10:
            c = kx[i] <= kx[j]
            kx[i], kx[j] = jnp.where(c, kx[i], kx[j]), jnp.where(c, kx[j], kx[i])
            ky[i], ky[j] = jnp.where(c, ky[i], ky[j]), jnp.where(c, ky[j], ky[i])
        return kx, ky

    sxx, sxy = net(xs_rows, ys_rows)
    syy, syx = net(ys_rows, xs_rows)

    feat = jnp.concatenate(
        [jnp.stack(sxx, 0), jnp.stack(sxy, 0),
         jnp.stack(syx, 0), jnp.stack(syy, 0), xsT], axis=0)

    out_ref[0] = jax.lax.dot_general(
        feat, m_ref[...], (((0,), (0,)), ((), ())),
        preferred_element_type=jnp.float32) + bias_ref[...]


def _build_M(conv_x_w, conv_y_w, W1_w, W2_w):
    Ax = conv_x_w.transpose(2, 1, 0).reshape(2 * K1, -1)
    Ay = conv_y_w.transpose(2, 1, 0).reshape(2 * K1, -1)
    Mx = Ax @ W2_w.T
    My = Ay @ W2_w.T
    E = conv_x_w.shape[0]
    M = jnp.concatenate([
        Mx[0::2],
        Mx[1::2],
        My[0::2],
        My[1::2],
        W1_w.T,
    ], axis=0)
    return M


@jax.jit
def kernel(x, conv_x_w, conv_x_b, conv_y_w, conv_y_b, W1_w, W1_b, W2_w, W2_b):
    B, N, _ = x.shape
    E = conv_x_w.shape[0]
    M = _build_M(conv_x_w, conv_y_w, W1_w, W2_w)
    bias = W1_b + (conv_x_b + conv_y_b) @ W2_w.T + W2_b
    xT = jnp.swapaxes(x, 1, 2)

    grid = (B, N // R)
    out = pl.pallas_call(
        _knn_body,
        grid=grid,
        in_specs=[
            pl.BlockSpec((1, 2, R), lambda b, j: (b, 0, j)),
            pl.BlockSpec((1, N, 2), lambda b, j: (b, 0, 0)),
            pl.BlockSpec((2 * K1 + 2 * K1 + 2, E), lambda b, j: (0, 0)),
            pl.BlockSpec((E,), lambda b, j: (0,)),
        ],
        out_specs=pl.BlockSpec((1, R, E), lambda b, j: (b, j, 0)),
        out_shape=jax.ShapeDtypeStruct((B, N, E), jnp.float32),
    )(xT, x, M, bias)
    return out


# speedup vs baseline: 57.7493x; 2.1506x over previous
import functools

import jax
import jax.numpy as jnp
from jax import lax
from jax.experimental import pallas as pl
from jax.experimental.pallas import tpu as pltpu
from jax.experimental.pallas import tpu_sc as plsc

K1 = 10
L = 16
BIG = 1e30
RBLK = 32
ROWI = 4


def _d2_body(x_ref, d2_ref):
    xa = x_ref[0]
    sq = jnp.sum(xa * xa, axis=1)
    g = jax.lax.dot_general(xa, xa, (((1,), (1,)), ((), ())),
                            preferred_element_type=jnp.float32)
    d2_ref[0] = jnp.sqrt(jnp.maximum(sq[:, None] + sq[None, :] - 2.0 * g, 0.0))


def _sc_body(n, rows_per_w, xx_hbm, xy_hbm, d2_hbm, feat_hbm,
             cx_v, cy_v, fb_v, db0_v, db1_v, sem0, sem1):
    wid = lax.axis_index("s") * 2 + lax.axis_index("c")
    base = wid * rows_per_w
    b = base // n
    n0 = base % n
    pltpu.sync_copy(xx_hbm.at[b], cx_v)
    pltpu.sync_copy(xy_hbm.at[b], cy_v)
    lane = lax.iota(jnp.int32, L)
    valid = lane < K1
    chunks = n // L
    nblk = rows_per_w // RBLK

    def fetch(blk, dbuf, sem):
        sl = pl.ds(base + blk * RBLK, RBLK)
        pltpu.async_copy(d2_hbm.at[sl], dbuf, sem)

    def wait_fetch(blk, dbuf, sem):
        sl = pl.ds(base + blk * RBLK, RBLK)
        pltpu.make_async_copy(d2_hbm.at[sl], dbuf, sem).wait()

    def process_rows(blk, rr, dbuf):
        states = []
        for i in range(ROWI):
            q = n0 + blk * RBLK + rr + i
            qidx = jnp.full((L,), q, jnp.int32)
            qxv = plsc.load_gather(cx_v, [qidx])
            qyv = plsc.load_gather(cy_v, [qidx])
            states.append((jnp.full((L,), BIG, jnp.float32),
                           jnp.zeros((L,), jnp.int32), qxv, qyv))

        def chunk_body(c, sts):
            off = (chunks - 1 - c) * L
            out = []
            for i in range(ROWI):
                ck, cv, qxv, qyv = sts[i]
                d2 = dbuf[rr + i, pl.ds(off, L)]
                dk, dv = plsc.sort_key_val(d2, lane + off, descending=True)
                sel = ck <= dk
                nk, nv = plsc.sort_key_val(jnp.where(sel, ck, dk),
                                           jnp.where(sel, cv, dv))
                out.append((nk, nv, qxv, qyv))
            return tuple(out)

        states = lax.fori_loop(0, chunks, chunk_body, tuple(states))

        for i in range(ROWI):
            r = blk * RBLK + rr + i
            ck, cv, qxv, qyv = states[i]
            gx = plsc.load_gather(cx_v, [cv])
            gy = plsc.load_gather(cy_v, [cv])
            kx = jnp.where(valid, gx, BIG)
            sxx, sxy = plsc.sort_key_val(kx, gy)
            ky = jnp.where(valid, gy, BIG)
            syy, syx = plsc.sort_key_val(ky, gx)
            sxx = jnp.where(lane == K1, qxv, sxx)
            sxx = jnp.where(lane == K1 + 1, qyv, sxx)
            fb_v[r, pl.ds(0 * L, L)] = sxx
            fb_v[r, pl.ds(1 * L, L)] = sxy
            fb_v[r, pl.ds(2 * L, L)] = syx
            fb_v[r, pl.ds(3 * L, L)] = syy

    fetch(0, db0_v, sem0)
    fetch(1, db1_v, sem1)

    def outer(blk2, carry):
        for par, dbuf, sem in ((0, db0_v, sem0), (1, db1_v, sem1)):
            blk = blk2 * 2 + par
            wait_fetch(blk, dbuf, sem)
            lax.fori_loop(
                0, RBLK // ROWI,
                lambda j, _, blk=blk, dbuf=dbuf:
                    (process_rows(blk, j * ROWI, dbuf), 0)[1], 0)

            @pl.when(blk + 2 < nblk)
            def _(blk=blk, dbuf=dbuf, sem=sem):
                fetch(blk + 2, dbuf, sem)
        return carry

    lax.fori_loop(0, nblk // 2, outer, 0)
    pltpu.sync_copy(fb_v, feat_hbm.at[pl.ds(base, rows_per_w)])


def _mm_body(f_ref, m_ref, bias_ref, out_ref):
    out_ref[...] = jax.lax.dot_general(
        f_ref[...], m_ref[...], (((1,), (0,)), ((), ())),
        preferred_element_type=jnp.float32) + bias_ref[...]


def _build_M64(conv_x_w, conv_y_w, W1_w, W2_w):
    Ax = conv_x_w.transpose(2, 1, 0).reshape(2 * K1, -1)
    Ay = conv_y_w.transpose(2, 1, 0).reshape(2 * K1, -1)
    Mx = Ax @ W2_w.T
    My = Ay @ W2_w.T
    E = conv_x_w.shape[0]
    M = jnp.zeros((4 * L, E), jnp.float32)
    M = M.at[0:K1].set(Mx[0::2])
    M = M.at[K1].set(W1_w[:, 0])
    M = M.at[K1 + 1].set(W1_w[:, 1])
    M = M.at[L:L + K1].set(Mx[1::2])
    M = M.at[2 * L:2 * L + K1].set(My[0::2])
    M = M.at[3 * L:3 * L + K1].set(My[1::2])
    return M


@jax.jit
def kernel(x, conv_x_w, conv_x_b, conv_y_w, conv_y_b, W1_w, W1_b, W2_w, W2_b):
    B, N, _ = x.shape
    E = conv_x_w.shape[0]
    NW = 32
    rows_per_w = B * N // NW

    M = _build_M64(conv_x_w, conv_y_w, W1_w, W2_w)
    bias = W1_b + (conv_x_b + conv_y_b) @ W2_w.T + W2_b
    xx = x[:, :, 0]
    xy = x[:, :, 1]

    d2 = pl.pallas_call(
        _d2_body,
        grid=(B,),
        in_specs=[pl.BlockSpec((1, N, 2), lambda b: (b, 0, 0))],
        out_specs=pl.BlockSpec((1, N, N), lambda b: (b, 0, 0)),
        out_shape=jax.ShapeDtypeStruct((B, N, N), jnp.float32),
    )(x).reshape(B * N, N)

    mesh = plsc.VectorSubcoreMesh(core_axis_name="c", subcore_axis_name="s")
    feat = pl.kernel(
        functools.partial(_sc_body, N, rows_per_w),
        out_type=jax.ShapeDtypeStruct((B * N, 4 * L), jnp.float32),
        mesh=mesh,
        scratch_types=[
            pltpu.VMEM((N,), jnp.float32),
            pltpu.VMEM((N,), jnp.float32),
            pltpu.VMEM((rows_per_w, 4 * L), jnp.float32),
            pltpu.VMEM((RBLK, N), jnp.float32),
            pltpu.VMEM((RBLK, N), jnp.float32),
            pltpu.SemaphoreType.DMA,
            pltpu.SemaphoreType.DMA,
        ],
        compiler_params=pltpu.CompilerParams(needs_layout_passes=False),
    )(xx, xy, d2)

    RT = 1024
    out = pl.pallas_call(
        _mm_body,
        grid=(B * N // RT,),
        in_specs=[
            pl.BlockSpec((RT, 4 * L), lambda i: (i, 0)),
            pl.BlockSpec((4 * L, E), lambda i: (0, 0)),
            pl.BlockSpec((E,), lambda i: (0,)),
        ],
        out_specs=pl.BlockSpec((RT, E), lambda i: (i, 0)),
        out_shape=jax.ShapeDtypeStruct((B * N, E), jnp.float32),
    )(feat, M, bias)
    return out.reshape(B, N, E)


# speedup vs baseline: 73.8006x; 1.2779x over previous
import functools

import jax
import jax.numpy as jnp
from jax import lax
from jax.experimental import pallas as pl
from jax.experimental.pallas import tpu as pltpu
from jax.experimental.pallas import tpu_sc as plsc

K1 = 10
L = 16
BIG = 1e30
RBLK = 32
ROWI = 8


def _d2_body(x_ref, d2_ref):
    xa = x_ref[0]
    sq = jnp.sum(xa * xa, axis=1)
    g = jax.lax.dot_general(xa, xa, (((1,), (1,)), ((), ())),
                            preferred_element_type=jnp.float32)
    d2_ref[0] = jnp.sqrt(jnp.maximum(sq[:, None] + sq[None, :] - 2.0 * g, 0.0))


def _sc_body(n, rows_per_w, xx_hbm, xy_hbm, d2_hbm, feat_hbm,
             cx_v, cy_v, fb_v, db0_v, db1_v, sem0, sem1):
    wid = lax.axis_index("s") * 2 + lax.axis_index("c")
    base = wid * rows_per_w
    b = base // n
    n0 = base % n
    pltpu.sync_copy(xx_hbm.at[b], cx_v)
    pltpu.sync_copy(xy_hbm.at[b], cy_v)
    lane = lax.iota(jnp.int32, L)
    valid = lane < K1
    chunks = n // L
    nblk = rows_per_w // RBLK

    def fetch(blk, dbuf, sem):
        sl = pl.ds(base + blk * RBLK, RBLK)
        pltpu.async_copy(d2_hbm.at[sl], dbuf, sem)

    def wait_fetch(blk, dbuf, sem):
        sl = pl.ds(base + blk * RBLK, RBLK)
        pltpu.make_async_copy(d2_hbm.at[sl], dbuf, sem).wait()

    def process_rows(blk, rr, dbuf):
        states = []
        for i in range(ROWI):
            states.append((jnp.full((L,), BIG, jnp.float32),
                           jnp.zeros((L,), jnp.int32)))

        def chunk_body(c, sts):
            off = (chunks - 1 - c) * L
            out = []
            for i in range(ROWI):
                ck, cv = sts[i]
                d2 = dbuf[rr + i, pl.ds(off, L)]
                dk, dv = plsc.sort_key_val(d2, lane + off, descending=True)
                sel = ck <= dk
                nk, nv = plsc.sort_key_val(jnp.where(sel, ck, dk),
                                           jnp.where(sel, cv, dv))
                out.append((nk, nv))
            return tuple(out)

        states = lax.fori_loop(0, chunks, chunk_body, tuple(states))

        for i in range(ROWI):
            r = blk * RBLK + rr + i
            ck, cv = states[i]
            q = n0 + r
            qidx = jnp.full((L,), q, jnp.int32)
            qxv = plsc.load_gather(cx_v, [qidx])
            qyv = plsc.load_gather(cy_v, [qidx])
            gx = plsc.load_gather(cx_v, [cv])
            gy = plsc.load_gather(cy_v, [cv])
            kx = jnp.where(valid, gx, BIG)
            sxx, sxy = plsc.sort_key_val(kx, gy)
            ky = jnp.where(valid, gy, BIG)
            syy, syx = plsc.sort_key_val(ky, gx)
            sxx = jnp.where(lane == K1, qxv, sxx)
            sxx = jnp.where(lane == K1 + 1, qyv, sxx)
            fb_v[r, pl.ds(0 * L, L)] = sxx
            fb_v[r, pl.ds(1 * L, L)] = sxy
            fb_v[r, pl.ds(2 * L, L)] = syx
            fb_v[r, pl.ds(3 * L, L)] = syy

    fetch(0, db0_v, sem0)
    fetch(1, db1_v, sem1)

    def outer(blk2, carry):
        for par, dbuf, sem in ((0, db0_v, sem0), (1, db1_v, sem1)):
            blk = blk2 * 2 + par
            wait_fetch(blk, dbuf, sem)
            lax.fori_loop(
                0, RBLK // ROWI,
                lambda j, _, blk=blk, dbuf=dbuf:
                    (process_rows(blk, j * ROWI, dbuf), 0)[1], 0)

            @pl.when(blk + 2 < nblk)
            def _(blk=blk, dbuf=dbuf, sem=sem):
                fetch(blk + 2, dbuf, sem)
        return carry

    lax.fori_loop(0, nblk // 2, outer, 0)
    pltpu.sync_copy(fb_v, feat_hbm.at[pl.ds(base, rows_per_w)])


def _mm_body(f_ref, m_ref, bias_ref, out_ref):
    out_ref[...] = jax.lax.dot_general(
        f_ref[...], m_ref[...], (((1,), (0,)), ((), ())),
        preferred_element_type=jnp.float32) + bias_ref[...]


def _build_M64(conv_x_w, conv_y_w, W1_w, W2_w):
    Ax = conv_x_w.transpose(2, 1, 0).reshape(2 * K1, -1)
    Ay = conv_y_w.transpose(2, 1, 0).reshape(2 * K1, -1)
    Mx = Ax @ W2_w.T
    My = Ay @ W2_w.T
    E = conv_x_w.shape[0]
    M = jnp.zeros((4 * L, E), jnp.float32)
    M = M.at[0:K1].set(Mx[0::2])
    M = M.at[K1].set(W1_w[:, 0])
    M = M.at[K1 + 1].set(W1_w[:, 1])
    M = M.at[L:L + K1].set(Mx[1::2])
    M = M.at[2 * L:2 * L + K1].set(My[0::2])
    M = M.at[3 * L:3 * L + K1].set(My[1::2])
    return M


@jax.jit
def kernel(x, conv_x_w, conv_x_b, conv_y_w, conv_y_b, W1_w, W1_b, W2_w, W2_b):
    B, N, _ = x.shape
    E = conv_x_w.shape[0]
    NW = 32
    rows_per_w = B * N // NW

    M = _build_M64(conv_x_w, conv_y_w, W1_w, W2_w)
    bias = W1_b + (conv_x_b + conv_y_b) @ W2_w.T + W2_b
    xx = x[:, :, 0]
    xy = x[:, :, 1]

    d2 = pl.pallas_call(
        _d2_body,
        grid=(B,),
        in_specs=[pl.BlockSpec((1, N, 2), lambda b: (b, 0, 0))],
        out_specs=pl.BlockSpec((1, N, N), lambda b: (b, 0, 0)),
        out_shape=jax.ShapeDtypeStruct((B, N, N), jnp.float32),
    )(x).reshape(B * N, N)

    mesh = plsc.VectorSubcoreMesh(core_axis_name="c", subcore_axis_name="s")
    feat = pl.kernel(
        functools.partial(_sc_body, N, rows_per_w),
        out_type=jax.ShapeDtypeStruct((B * N, 4 * L), jnp.float32),
        mesh=mesh,
        scratch_types=[
            pltpu.VMEM((N,), jnp.float32),
            pltpu.VMEM((N,), jnp.float32),
            pltpu.VMEM((rows_per_w, 4 * L), jnp.float32),
            pltpu.VMEM((RBLK, N), jnp.float32),
            pltpu.VMEM((RBLK, N), jnp.float32),
            pltpu.SemaphoreType.DMA,
            pltpu.SemaphoreType.DMA,
        ],
        compiler_params=pltpu.CompilerParams(needs_layout_passes=False),
    )(xx, xy, d2)

    RT = 1024
    out = pl.pallas_call(
        _mm_body,
        grid=(B * N // RT,),
        in_specs=[
            pl.BlockSpec((RT, 4 * L), lambda i: (i, 0)),
            pl.BlockSpec((4 * L, E), lambda i: (0, 0)),
            pl.BlockSpec((E,), lambda i: (0,)),
        ],
        out_specs=pl.BlockSpec((RT, E), lambda i: (i, 0)),
        out_shape=jax.ShapeDtypeStruct((B * N, E), jnp.float32),
    )(feat, M, bias)
    return out.reshape(B, N, E)


# speedup vs baseline: 78.1974x; 1.0596x over previous
import functools

import jax
import jax.numpy as jnp
from jax import lax
from jax.experimental import pallas as pl
from jax.experimental.pallas import tpu as pltpu
from jax.experimental.pallas import tpu_sc as plsc

K1 = 10
L = 16
BIG = 1e30
RBLK = 32
ROWI = 16


def _d2_body(x_ref, d2_ref):
    xa = x_ref[0]
    sq = jnp.sum(xa * xa, axis=1)
    g = jax.lax.dot_general(xa, xa, (((1,), (1,)), ((), ())),
                            preferred_element_type=jnp.float32)
    d2_ref[0] = jnp.sqrt(jnp.maximum(sq[:, None] + sq[None, :] - 2.0 * g, 0.0))


def _sc_body(n, rows_per_w, xx_hbm, xy_hbm, d2_hbm, feat_hbm,
             cx_v, cy_v, fb_v, db0_v, db1_v, sem0, sem1):
    wid = lax.axis_index("s") * 2 + lax.axis_index("c")
    base = wid * rows_per_w
    b = base // n
    n0 = base % n
    pltpu.sync_copy(xx_hbm.at[b], cx_v)
    pltpu.sync_copy(xy_hbm.at[b], cy_v)
    lane = lax.iota(jnp.int32, L)
    valid = lane < K1
    chunks = n // L
    nblk = rows_per_w // RBLK

    def fetch(blk, dbuf, sem):
        sl = pl.ds(base + blk * RBLK, RBLK)
        pltpu.async_copy(d2_hbm.at[sl], dbuf, sem)

    def wait_fetch(blk, dbuf, sem):
        sl = pl.ds(base + blk * RBLK, RBLK)
        pltpu.make_async_copy(d2_hbm.at[sl], dbuf, sem).wait()

    def process_rows(blk, rr, dbuf):
        states = []
        for i in range(ROWI):
            states.append((jnp.full((L,), BIG, jnp.float32),
                           jnp.zeros((L,), jnp.int32)))

        def chunk_body(c, sts):
            off = (chunks - 1 - c) * L
            out = []
            for i in range(ROWI):
                ck, cv = sts[i]
                d2 = dbuf[rr + i, pl.ds(off, L)]
                dk, dv = plsc.sort_key_val(d2, lane + off, descending=True)
                sel = ck <= dk
                nk, nv = plsc.sort_key_val(jnp.where(sel, ck, dk),
                                           jnp.where(sel, cv, dv))
                out.append((nk, nv))
            return tuple(out)

        states = lax.fori_loop(0, chunks, chunk_body, tuple(states))

        for i in range(ROWI):
            r = blk * RBLK + rr + i
            ck, cv = states[i]
            q = n0 + r
            qidx = jnp.full((L,), q, jnp.int32)
            qxv = plsc.load_gather(cx_v, [qidx])
            qyv = plsc.load_gather(cy_v, [qidx])
            gx = plsc.load_gather(cx_v, [cv])
            gy = plsc.load_gather(cy_v, [cv])
            kx = jnp.where(valid, gx, BIG)
            sxx, sxy = plsc.sort_key_val(kx, gy)
            ky = jnp.where(valid, gy, BIG)
            syy, syx = plsc.sort_key_val(ky, gx)
            sxx = jnp.where(lane == K1, qxv, sxx)
            sxx = jnp.where(lane == K1 + 1, qyv, sxx)
            fb_v[r, pl.ds(0 * L, L)] = sxx
            fb_v[r, pl.ds(1 * L, L)] = sxy
            fb_v[r, pl.ds(2 * L, L)] = syx
            fb_v[r, pl.ds(3 * L, L)] = syy

    fetch(0, db0_v, sem0)
    fetch(1, db1_v, sem1)

    def outer(blk2, carry):
        for par, dbuf, sem in ((0, db0_v, sem0), (1, db1_v, sem1)):
            blk = blk2 * 2 + par
            wait_fetch(blk, dbuf, sem)
            lax.fori_loop(
                0, RBLK // ROWI,
                lambda j, _, blk=blk, dbuf=dbuf:
                    (process_rows(blk, j * ROWI, dbuf), 0)[1], 0)

            @pl.when(blk + 2 < nblk)
            def _(blk=blk, dbuf=dbuf, sem=sem):
                fetch(blk + 2, dbuf, sem)
        return carry

    lax.fori_loop(0, nblk // 2, outer, 0)
    pltpu.sync_copy(fb_v, feat_hbm.at[pl.ds(base, rows_per_w)])


def _mm_body(f_ref, m_ref, bias_ref, out_ref):
    out_ref[...] = jax.lax.dot_general(
        f_ref[...], m_ref[...], (((1,), (0,)), ((), ())),
        preferred_element_type=jnp.float32) + bias_ref[...]


def _build_M64(conv_x_w, conv_y_w, W1_w, W2_w):
    Ax = conv_x_w.transpose(2, 1, 0).reshape(2 * K1, -1)
    Ay = conv_y_w.transpose(2, 1, 0).reshape(2 * K1, -1)
    Mx = Ax @ W2_w.T
    My = Ay @ W2_w.T
    E = conv_x_w.shape[0]
    M = jnp.zeros((4 * L, E), jnp.float32)
    M = M.at[0:K1].set(Mx[0::2])
    M = M.at[K1].set(W1_w[:, 0])
    M = M.at[K1 + 1].set(W1_w[:, 1])
    M = M.at[L:L + K1].set(Mx[1::2])
    M = M.at[2 * L:2 * L + K1].set(My[0::2])
    M = M.at[3 * L:3 * L + K1].set(My[1::2])
    return M


@jax.jit
def kernel(x, conv_x_w, conv_x_b, conv_y_w, conv_y_b, W1_w, W1_b, W2_w, W2_b):
    B, N, _ = x.shape
    E = conv_x_w.shape[0]
    NW = 32
    rows_per_w = B * N // NW

    M = _build_M64(conv_x_w, conv_y_w, W1_w, W2_w)
    bias = W1_b + (conv_x_b + conv_y_b) @ W2_w.T + W2_b
    xx = x[:, :, 0]
    xy = x[:, :, 1]

    d2 = pl.pallas_call(
        _d2_body,
        grid=(B,),
        in_specs=[pl.BlockSpec((1, N, 2), lambda b: (b, 0, 0))],
        out_specs=pl.BlockSpec((1, N, N), lambda b: (b, 0, 0)),
        out_shape=jax.ShapeDtypeStruct((B, N, N), jnp.float32),
    )(x).reshape(B * N, N)

    mesh = plsc.VectorSubcoreMesh(core_axis_name="c", subcore_axis_name="s")
    feat = pl.kernel(
        functools.partial(_sc_body, N, rows_per_w),
        out_type=jax.ShapeDtypeStruct((B * N, 4 * L), jnp.float32),
        mesh=mesh,
        scratch_types=[
            pltpu.VMEM((N,), jnp.float32),
            pltpu.VMEM((N,), jnp.float32),
            pltpu.VMEM((rows_per_w, 4 * L), jnp.float32),
            pltpu.VMEM((RBLK, N), jnp.float32),
            pltpu.VMEM((RBLK, N), jnp.float32),
            pltpu.SemaphoreType.DMA,
            pltpu.SemaphoreType.DMA,
        ],
        compiler_params=pltpu.CompilerParams(needs_layout_passes=False),
    )(xx, xy, d2)

    RT = 1024
    out = pl.pallas_call(
        _mm_body,
        grid=(B * N // RT,),
        in_specs=[
            pl.BlockSpec((RT, 4 * L), lambda i: (i, 0)),
            pl.BlockSpec((4 * L, E), lambda i: (0, 0)),
            pl.BlockSpec((E,), lambda i: (0,)),
        ],
        out_specs=pl.BlockSpec((RT, E), lambda i: (i, 0)),
        out_shape=jax.ShapeDtypeStruct((B * N, E), jnp.float32),
    )(feat, M, bias)
    return out.reshape(B, N, E)
